# Initial kernel scaffold; baseline (speedup 1.0000x reference)
import jax
import jax.numpy as jnp
from jax.experimental import pallas as pl


def kernel(center_words, target_words, negative_words, Wv, Wu):
    raise NotImplementedError("write your pallas kernel here")


# speedup vs baseline: 2.5657x; 2.5657x over previous
import functools

import jax
import jax.numpy as jnp
from jax import lax
from jax.experimental import pallas as pl
from jax.experimental.pallas import tpu as pltpu
from jax.experimental.pallas import tpu_sc as plsc

_VOCAB = 100000
_D = 128
_B = 4096
_NEG = 20
_NC = 2
_NS = 16
_NW = _NC * _NS
_NB = _B // _NW


def _sc_gather(center, target, neg_t, wv, wu):
    mesh = plsc.VectorSubcoreMesh(core_axis_name="c", subcore_axis_name="s")

    @functools.partial(
        pl.kernel,
        out_type=[jax.ShapeDtypeStruct((_B, _D), jnp.float32)] * 3,
        mesh=mesh,
        scratch_types=[
            pltpu.VMEM((_NB,), jnp.int32),
            pltpu.VMEM((_NB,), jnp.int32),
            pltpu.VMEM((_NEG, _NB), jnp.int32),
            pltpu.VMEM((_NB, _D), jnp.float32),
            pltpu.VMEM((_NB, _D), jnp.float32),
            pltpu.VMEM((_NB, _D), jnp.float32),
            pltpu.SemaphoreType.DMA,
            pltpu.SemaphoreType.DMA,
            pltpu.SemaphoreType.DMA,
        ],
    )
    def k(center_hbm, target_hbm, negt_hbm, wv_hbm, wu_hbm,
          c_out, t_out, s_out,
          cidx, tidx, nidx, cbuf, tbuf, sbuf, sem_c, sem_t, sem_s):
        wid = lax.axis_index("s") * _NC + lax.axis_index("c")
        base = wid * _NB
        pltpu.sync_copy(center_hbm.at[pl.ds(base, _NB)], cidx)
        pltpu.sync_copy(target_hbm.at[pl.ds(base, _NB)], tidx)
        pltpu.sync_copy(negt_hbm.at[:, pl.ds(base, _NB)], nidx)
        cg = pltpu.async_copy(wv_hbm.at[cidx], cbuf, sem_c)
        tg = pltpu.async_copy(wu_hbm.at[tidx], tbuf, sem_t)
        pltpu.async_copy(wu_hbm.at[nidx.at[0]], sbuf, sem_s).wait()
        for kk in range(1, _NEG):
            pltpu.async_copy(wu_hbm.at[nidx.at[kk]], sbuf, sem_s, add=True).wait()
        cg.wait()
        tg.wait()
        pltpu.sync_copy(cbuf, c_out.at[pl.ds(base, _NB)])
        pltpu.sync_copy(tbuf, t_out.at[pl.ds(base, _NB)])
        pltpu.sync_copy(sbuf, s_out.at[pl.ds(base, _NB)])

    return k(center, target, neg_t, wv, wu)


def _log_sigmoid(x):
    return jnp.minimum(x, 0.0) - jnp.log1p(jnp.exp(-jnp.abs(x)))


def _tc_loss_body(c_ref, t_ref, s_ref, out_ref):
    c = c_ref[...]
    pos = jnp.sum(c * t_ref[...], axis=1, keepdims=True)
    neg = -jnp.sum(c * s_ref[...], axis=1, keepdims=True)
    los = _log_sigmoid(pos) + _log_sigmoid(neg)
    out_ref[0, 0] = -jnp.sum(los) / _B


def _tc_loss(c_rows, t_rows, s_rows):
    out = pl.pallas_call(
        _tc_loss_body,
        out_shape=jax.ShapeDtypeStruct((1, 1), jnp.float32),
        out_specs=pl.BlockSpec(memory_space=pltpu.SMEM),
    )(c_rows, t_rows, s_rows)
    return out[0, 0]


def kernel(center_words, target_words, negative_words, Wv, Wu):
    center = center_words.reshape(_B).astype(jnp.int32)
    target = target_words.reshape(_B).astype(jnp.int32)
    neg_t = negative_words.astype(jnp.int32).T
    c_rows, t_rows, s_rows = _sc_gather(center, target, neg_t, Wv, Wu)
    return _tc_loss(c_rows, t_rows, s_rows)


# speedup vs baseline: 3.2035x; 1.2486x over previous
import functools

import jax
import jax.numpy as jnp
from jax import lax
from jax.experimental import pallas as pl
from jax.experimental.pallas import tpu as pltpu
from jax.experimental.pallas import tpu_sc as plsc

_VOCAB = 100000
_D = 128
_B = 4096
_NEG = 20
_NC = 2
_NS = 16
_NW = _NC * _NS
_NB = _B // _NW


def _sc_gather(center, target, neg_t, wv, wu):
    mesh = plsc.VectorSubcoreMesh(core_axis_name="c", subcore_axis_name="s")

    @functools.partial(
        pl.kernel,
        out_type=[jax.ShapeDtypeStruct((_B, _D), jnp.float32)] * 3,
        mesh=mesh,
        scratch_types=[
            pltpu.VMEM((_NB,), jnp.int32),
            pltpu.VMEM((_NB,), jnp.int32),
            pltpu.VMEM((_NEG, _NB), jnp.int32),
            pltpu.VMEM((_NB, _D), jnp.float32),
            pltpu.VMEM((_NB, _D), jnp.float32),
            pltpu.VMEM((_NB, _D), jnp.float32),
            pltpu.SemaphoreType.DMA,
            pltpu.SemaphoreType.DMA,
            pltpu.SemaphoreType.DMA,
        ],
    )
    def k(center_hbm, target_hbm, negt_hbm, wv_hbm, wu_hbm,
          c_out, t_out, s_out,
          cidx, tidx, nidx, cbuf, tbuf, sbuf, sem_c, sem_t, sem_s):
        wid = lax.axis_index("s") * _NC + lax.axis_index("c")
        base = wid * _NB
        pltpu.sync_copy(center_hbm.at[pl.ds(base, _NB)], cidx)
        pltpu.sync_copy(target_hbm.at[pl.ds(base, _NB)], tidx)
        pltpu.sync_copy(negt_hbm.at[:, pl.ds(base, _NB)], nidx)
        cg = pltpu.async_copy(wv_hbm.at[cidx], cbuf, sem_c)
        tg = pltpu.async_copy(wu_hbm.at[tidx], tbuf, sem_t)
        pltpu.async_copy(wu_hbm.at[nidx.at[0]], sbuf, sem_s).wait()
        adds = [
            pltpu.async_copy(wu_hbm.at[nidx.at[kk]], sbuf, sem_s, add=True)
            for kk in range(1, _NEG)
        ]
        for a in adds:
            a.wait()
        cg.wait()
        tg.wait()
        pltpu.sync_copy(cbuf, c_out.at[pl.ds(base, _NB)])
        pltpu.sync_copy(tbuf, t_out.at[pl.ds(base, _NB)])
        pltpu.sync_copy(sbuf, s_out.at[pl.ds(base, _NB)])

    return k(center, target, neg_t, wv, wu)


def _log_sigmoid(x):
    return jnp.minimum(x, 0.0) - jnp.log1p(jnp.exp(-jnp.abs(x)))


def _tc_loss_body(c_ref, t_ref, s_ref, out_ref):
    c = c_ref[...]
    pos = jnp.sum(c * t_ref[...], axis=1, keepdims=True)
    neg = -jnp.sum(c * s_ref[...], axis=1, keepdims=True)
    los = _log_sigmoid(pos) + _log_sigmoid(neg)
    out_ref[0, 0] = -jnp.sum(los) / _B


def _tc_loss(c_rows, t_rows, s_rows):
    out = pl.pallas_call(
        _tc_loss_body,
        out_shape=jax.ShapeDtypeStruct((1, 1), jnp.float32),
        out_specs=pl.BlockSpec(memory_space=pltpu.SMEM),
    )(c_rows, t_rows, s_rows)
    return out[0, 0]


def kernel(center_words, target_words, negative_words, Wv, Wu):
    center = center_words.reshape(_B).astype(jnp.int32)
    target = target_words.reshape(_B).astype(jnp.int32)
    neg_t = negative_words.astype(jnp.int32).T
    c_rows, t_rows, s_rows = _sc_gather(center, target, neg_t, Wv, Wu)
    return _tc_loss(c_rows, t_rows, s_rows)


# speedup vs baseline: 3.2257x; 1.0069x over previous
import functools

import jax
import jax.numpy as jnp
from jax import lax
from jax.experimental import pallas as pl
from jax.experimental.pallas import tpu as pltpu
from jax.experimental.pallas import tpu_sc as plsc

_VOCAB = 100000
_D = 128
_B = 4096
_NEG = 20
_NC = 2
_NS = 16
_NW = _NC * _NS
_NB = _B // _NW
_L = 16


def _sc_gather_score(center, target, neg_t, wv, wu):
    mesh = plsc.VectorSubcoreMesh(core_axis_name="c", subcore_axis_name="s")

    @functools.partial(
        pl.kernel,
        out_type=[jax.ShapeDtypeStruct((_B, _L), jnp.float32)] * 2,
        mesh=mesh,
        scratch_types=[
            pltpu.VMEM((_NB,), jnp.int32),
            pltpu.VMEM((_NB,), jnp.int32),
            pltpu.VMEM((_NEG, _NB), jnp.int32),
            pltpu.VMEM((_NB, _D), jnp.float32),
            pltpu.VMEM((_NB, _D), jnp.float32),
            pltpu.VMEM((_NB, _D), jnp.float32),
            pltpu.VMEM((_NB, _L), jnp.float32),
            pltpu.VMEM((_NB, _L), jnp.float32),
            pltpu.SemaphoreType.DMA,
            pltpu.SemaphoreType.DMA,
            pltpu.SemaphoreType.DMA,
        ],
    )
    def k(center_hbm, target_hbm, negt_hbm, wv_hbm, wu_hbm,
          pos_out, neg_out,
          cidx, tidx, nidx, cbuf, tbuf, sbuf, pdot, ndot,
          sem_c, sem_t, sem_s):
        wid = lax.axis_index("s") * _NC + lax.axis_index("c")
        base = wid * _NB
        pltpu.sync_copy(center_hbm.at[pl.ds(base, _NB)], cidx)
        pltpu.sync_copy(target_hbm.at[pl.ds(base, _NB)], tidx)
        pltpu.sync_copy(negt_hbm.at[:, pl.ds(base, _NB)], nidx)
        cg = pltpu.async_copy(wv_hbm.at[cidx], cbuf, sem_c)
        tg = pltpu.async_copy(wu_hbm.at[tidx], tbuf, sem_t)
        ng0 = pltpu.async_copy(wu_hbm.at[nidx.at[0]], sbuf, sem_s)
        cg.wait()
        tg.wait()
        ng0.wait()
        adds = [
            pltpu.async_copy(wu_hbm.at[nidx.at[kk]], sbuf, sem_s, add=True)
            for kk in range(1, _NEG)
        ]

        def dot_partial(buf_a, buf_b, b):
            acc = buf_a[b, pl.ds(0, _L)] * buf_b[b, pl.ds(0, _L)]
            for j in range(1, _D // _L):
                acc += buf_a[b, pl.ds(j * _L, _L)] * buf_b[b, pl.ds(j * _L, _L)]
            return acc

        def pos_body(b, carry):
            pdot[b, :] = dot_partial(cbuf, tbuf, b)
            return carry

        lax.fori_loop(0, _NB, pos_body, 0)
        for a in adds:
            a.wait()

        def neg_body(b, carry):
            ndot[b, :] = dot_partial(cbuf, sbuf, b)
            return carry

        lax.fori_loop(0, _NB, neg_body, 0)
        pltpu.sync_copy(pdot, pos_out.at[pl.ds(base, _NB), :])
        pltpu.sync_copy(ndot, neg_out.at[pl.ds(base, _NB), :])

    return k(center, target, neg_t, wv, wu)


def _log_sigmoid(x):
    return jnp.minimum(x, 0.0) - jnp.log1p(jnp.exp(-jnp.abs(x)))


def _tc_loss_body(p_ref, n_ref, out_ref):
    pos = jnp.sum(p_ref[...], axis=1)
    neg = -jnp.sum(n_ref[...], axis=1)
    los = _log_sigmoid(pos) + _log_sigmoid(neg)
    out_ref[0, 0] = -jnp.sum(los) / _B


def _tc_loss(pos_part, neg_part):
    out = pl.pallas_call(
        _tc_loss_body,
        out_shape=jax.ShapeDtypeStruct((1, 1), jnp.float32),
        out_specs=pl.BlockSpec(memory_space=pltpu.SMEM),
    )(pos_part, neg_part)
    return out[0, 0]


def kernel(center_words, target_words, negative_words, Wv, Wu):
    center = center_words.reshape(_B).astype(jnp.int32)
    target = target_words.reshape(_B).astype(jnp.int32)
    neg_t = negative_words.astype(jnp.int32).T
    pos_part, neg_part = _sc_gather_score(center, target, neg_t, Wv, Wu)
    return _tc_loss(pos_part, neg_part)
